# baseline (speedup 1.0000x reference)
import functools
import math

import jax
import jax.numpy as jnp
from jax import lax
from jax.experimental import pallas as pl
from jax.experimental.pallas import tpu as pltpu
from jax.experimental.pallas import tpu_sc as plsc

VOCAB = 100000
NUM_FREQ = 50
EMBED_DIM = 64
TROW = 128
ROW_BLOCK = 2000

CHUNK = 128
NBUF = 4


FGROUPS = 7


def _tc_table_body(a_ref, b_ref, wt_ref, bias_ref, out_ref):
    i = pl.program_id(0)
    vf = (lax.broadcasted_iota(jnp.int32, (8, ROW_BLOCK), 1)
          + i * ROW_BLOCK).astype(jnp.float32)
    phi = (2.0 * math.pi) * (vf / float(VOCAB))
    f0 = (lax.broadcasted_iota(jnp.int32, (8, ROW_BLOCK), 0) + 1
          ).astype(jnp.float32)
    arg = f0 * phi
    c = jnp.cos(arg)
    s = jnp.sin(arg)
    cr = jnp.cos(8.0 * phi)
    sr = jnp.sin(8.0 * phi)
    cs, ss = [c], [s]
    for _ in range(FGROUPS - 1):
        c, s = c * cr - s * sr, c * sr + s * cr
        cs.append(c)
        ss.append(s)
    ctab = jnp.concatenate(cs, axis=0)
    stab = jnp.concatenate(ss, axis=0)
    ct = jnp.transpose(ctab)[:, 0:NUM_FREQ]
    st = jnp.transpose(stab)[:, 0:NUM_FREQ]
    emb = a_ref[...] * ct + b_ref[...] * st
    out_ref[:, 0:EMBED_DIM] = (
        jnp.dot(emb, wt_ref[...], preferred_element_type=jnp.float32)
        + bias_ref[0:1, :]
    )
    out_ref[:, EMBED_DIM:TROW] = jnp.zeros(
        (ROW_BLOCK, TROW - EMBED_DIM), jnp.float32)


def _build_table(a_n, b_n, wt, bias_blk):
    grid = VOCAB // ROW_BLOCK
    return pl.pallas_call(
        _tc_table_body,
        grid=(grid,),
        in_specs=[
            pl.BlockSpec((ROW_BLOCK, NUM_FREQ), lambda i: (i, 0)),
            pl.BlockSpec((ROW_BLOCK, NUM_FREQ), lambda i: (i, 0)),
            pl.BlockSpec((NUM_FREQ, EMBED_DIM), lambda i: (0, 0)),
            pl.BlockSpec((8, EMBED_DIM), lambda i: (0, 0)),
        ],
        out_specs=pl.BlockSpec((ROW_BLOCK, TROW), lambda i: (i, 0)),
        out_shape=jax.ShapeDtypeStruct((VOCAB, TROW), jnp.float32),
    )(a_n, b_n, wt, bias_blk)


def _sc_gather(ids2d, table):
    n_rows = ids2d.shape[0]
    ntok = n_rows * CHUNK
    info = plsc.get_sparse_core_info()
    nc, ns = info.num_cores, info.num_subcores
    nw = nc * ns
    rows_per_w = n_rows // nw
    outer = rows_per_w // NBUF

    mesh = plsc.VectorSubcoreMesh(core_axis_name="c", subcore_axis_name="s")

    @functools.partial(
        pl.kernel,
        mesh=mesh,
        out_type=jax.ShapeDtypeStruct((ntok, TROW), jnp.float32),
        scratch_types=[
            pltpu.VMEM((rows_per_w, CHUNK), jnp.int32),
            pltpu.VMEM((NBUF, CHUNK, TROW), jnp.float32),
            [pltpu.SemaphoreType.DMA] * NBUF,
            [pltpu.SemaphoreType.DMA] * NBUF,
        ],
    )
    def k(ids_hbm, table_hbm, out_hbm, idx_v, rows_v, gsems, wsems):
        wid = lax.axis_index("s") * nc + lax.axis_index("c")
        row0 = wid * rows_per_w
        tok0 = row0 * CHUNK
        pltpu.sync_copy(ids_hbm.at[pl.ds(row0, rows_per_w)], idx_v)

        def fire(s, buf):
            pltpu.async_copy(table_hbm.at[idx_v.at[s]],
                             rows_v.at[buf], gsems[buf])

        def drain_g(buf):
            pltpu.make_async_copy(table_hbm.at[idx_v.at[0]],
                                  rows_v.at[buf], gsems[buf]).wait()

        def write(s, buf):
            pltpu.async_copy(rows_v.at[buf],
                             out_hbm.at[pl.ds(tok0 + s * CHUNK, CHUNK)],
                             wsems[buf])

        def drain_w(buf):
            pltpu.make_async_copy(rows_v.at[buf],
                                  out_hbm.at[pl.ds(tok0, CHUNK)],
                                  wsems[buf]).wait()

        for b in range(NBUF - 1):
            fire(b, b)

        def body(h, carry):
            for b in range(NBUF):
                s = NBUF * h + b
                nxt = s + NBUF - 1
                nbuf = (b + NBUF - 1) % NBUF

                @pl.when(jnp.logical_and(nxt >= NBUF, nxt < rows_per_w))
                def _():
                    drain_w(nbuf)

                @pl.when(nxt < rows_per_w)
                def _():
                    fire(nxt, nbuf)
                drain_g(b)
                write(s, b)
            return carry

        lax.fori_loop(0, outer, body, 0)
        for b in range(NBUF):
            drain_w(b)

    return k(ids2d, table)


def kernel(token_ids, a_n, b_n, W, b):
    B, S = token_ids.shape
    wt = W.T
    bias_blk = jnp.broadcast_to(b.reshape(1, EMBED_DIM), (8, EMBED_DIM))
    table = _build_table(a_n, b_n, wt, bias_blk)
    ids2d = token_ids.reshape(-1, CHUNK).astype(jnp.int32)
    out128 = _sc_gather(ids2d, table)
    return out128[:, 0:EMBED_DIM].reshape(B, S, EMBED_DIM)


# speedup vs baseline: 1.0241x; 1.0241x over previous
import functools
import math

import jax
import jax.numpy as jnp
from jax import lax
from jax.experimental import pallas as pl
from jax.experimental.pallas import tpu as pltpu
from jax.experimental.pallas import tpu_sc as plsc

VOCAB = 100000
NUM_FREQ = 50
EMBED_DIM = 64
TROW = 128
ROW_BLOCK = 4000

CHUNK = 128
NBUF = 4


FGROUPS = 7


def _tc_table_body(a_ref, b_ref, wt_ref, bias_ref, out_ref):
    i = pl.program_id(0)
    vf = (lax.broadcasted_iota(jnp.int32, (8, ROW_BLOCK), 1)
          + i * ROW_BLOCK).astype(jnp.float32)
    phi = (2.0 * math.pi) * (vf / float(VOCAB))
    f0 = (lax.broadcasted_iota(jnp.int32, (8, ROW_BLOCK), 0) + 1
          ).astype(jnp.float32)
    arg = f0 * phi
    c = jnp.cos(arg)
    s = jnp.sin(arg)
    cr = jnp.cos(8.0 * phi)
    sr = jnp.sin(8.0 * phi)
    cs, ss = [c], [s]
    for _ in range(FGROUPS - 1):
        c, s = c * cr - s * sr, c * sr + s * cr
        cs.append(c)
        ss.append(s)
    ctab = jnp.concatenate(cs, axis=0)
    stab = jnp.concatenate(ss, axis=0)
    ct = jnp.transpose(ctab)[:, 0:NUM_FREQ]
    st = jnp.transpose(stab)[:, 0:NUM_FREQ]
    emb = a_ref[...] * ct + b_ref[...] * st
    out_ref[:, 0:EMBED_DIM] = (
        jnp.dot(emb, wt_ref[...], preferred_element_type=jnp.float32)
        + bias_ref[0:1, :]
    )
    out_ref[:, EMBED_DIM:TROW] = jnp.zeros(
        (ROW_BLOCK, TROW - EMBED_DIM), jnp.float32)


def _build_table(a_n, b_n, wt, bias_blk):
    grid = VOCAB // ROW_BLOCK
    return pl.pallas_call(
        _tc_table_body,
        grid=(grid,),
        in_specs=[
            pl.BlockSpec((ROW_BLOCK, NUM_FREQ), lambda i: (i, 0)),
            pl.BlockSpec((ROW_BLOCK, NUM_FREQ), lambda i: (i, 0)),
            pl.BlockSpec((NUM_FREQ, EMBED_DIM), lambda i: (0, 0)),
            pl.BlockSpec((8, EMBED_DIM), lambda i: (0, 0)),
        ],
        out_specs=pl.BlockSpec((ROW_BLOCK, TROW), lambda i: (i, 0)),
        out_shape=jax.ShapeDtypeStruct((VOCAB, TROW), jnp.float32),
    )(a_n, b_n, wt, bias_blk)


def _sc_gather(ids2d, table):
    n_rows = ids2d.shape[0]
    ntok = n_rows * CHUNK
    info = plsc.get_sparse_core_info()
    nc, ns = info.num_cores, info.num_subcores
    nw = nc * ns
    rows_per_w = n_rows // nw
    outer = rows_per_w // NBUF

    mesh = plsc.VectorSubcoreMesh(core_axis_name="c", subcore_axis_name="s")

    @functools.partial(
        pl.kernel,
        mesh=mesh,
        out_type=jax.ShapeDtypeStruct((ntok, TROW), jnp.float32),
        scratch_types=[
            pltpu.VMEM((rows_per_w, CHUNK), jnp.int32),
            pltpu.VMEM((NBUF, CHUNK, TROW), jnp.float32),
            [pltpu.SemaphoreType.DMA] * NBUF,
            [pltpu.SemaphoreType.DMA] * NBUF,
        ],
    )
    def k(ids_hbm, table_hbm, out_hbm, idx_v, rows_v, gsems, wsems):
        wid = lax.axis_index("s") * nc + lax.axis_index("c")
        row0 = wid * rows_per_w
        tok0 = row0 * CHUNK
        pltpu.sync_copy(ids_hbm.at[pl.ds(row0, rows_per_w)], idx_v)

        def fire(s, buf):
            pltpu.async_copy(table_hbm.at[idx_v.at[s]],
                             rows_v.at[buf], gsems[buf])

        def drain_g(buf):
            pltpu.make_async_copy(table_hbm.at[idx_v.at[0]],
                                  rows_v.at[buf], gsems[buf]).wait()

        def write(s, buf):
            pltpu.async_copy(rows_v.at[buf],
                             out_hbm.at[pl.ds(tok0 + s * CHUNK, CHUNK)],
                             wsems[buf])

        def drain_w(buf):
            pltpu.make_async_copy(rows_v.at[buf],
                                  out_hbm.at[pl.ds(tok0, CHUNK)],
                                  wsems[buf]).wait()

        for b in range(NBUF - 1):
            fire(b, b)

        def body(h, carry):
            for b in range(NBUF):
                s = NBUF * h + b
                nxt = s + NBUF - 1
                nbuf = (b + NBUF - 1) % NBUF

                @pl.when(jnp.logical_and(nxt >= NBUF, nxt < rows_per_w))
                def _():
                    drain_w(nbuf)

                @pl.when(nxt < rows_per_w)
                def _():
                    fire(nxt, nbuf)
                drain_g(b)
                write(s, b)
            return carry

        lax.fori_loop(0, outer, body, 0)
        for b in range(NBUF):
            drain_w(b)

    return k(ids2d, table)


def kernel(token_ids, a_n, b_n, W, b):
    B, S = token_ids.shape
    wt = W.T
    bias_blk = jnp.broadcast_to(b.reshape(1, EMBED_DIM), (8, EMBED_DIM))
    table = _build_table(a_n, b_n, wt, bias_blk)
    ids2d = token_ids.reshape(-1, CHUNK).astype(jnp.int32)
    out128 = _sc_gather(ids2d, table)
    return out128[:, 0:EMBED_DIM].reshape(B, S, EMBED_DIM)


# speedup vs baseline: 1.0336x; 1.0092x over previous
import functools
import math

import jax
import jax.numpy as jnp
from jax import lax
from jax.experimental import pallas as pl
from jax.experimental.pallas import tpu as pltpu
from jax.experimental.pallas import tpu_sc as plsc

VOCAB = 100000
NUM_FREQ = 50
EMBED_DIM = 64
TROW = 128
ROW_BLOCK = 5000

CHUNK = 128
NBUF = 4


FGROUPS = 7


def _tc_table_body(a_ref, b_ref, wt_ref, bias_ref, out_ref):
    i = pl.program_id(0)
    vf = (lax.broadcasted_iota(jnp.int32, (8, ROW_BLOCK), 1)
          + i * ROW_BLOCK).astype(jnp.float32)
    phi = (2.0 * math.pi) * (vf / float(VOCAB))
    f0 = (lax.broadcasted_iota(jnp.int32, (8, ROW_BLOCK), 0) + 1
          ).astype(jnp.float32)
    arg = f0 * phi
    c = jnp.cos(arg)
    s = jnp.sin(arg)
    cr = jnp.cos(8.0 * phi)
    sr = jnp.sin(8.0 * phi)
    cs, ss = [c], [s]
    for _ in range(FGROUPS - 1):
        c, s = c * cr - s * sr, c * sr + s * cr
        cs.append(c)
        ss.append(s)
    ctab = jnp.concatenate(cs, axis=0)
    stab = jnp.concatenate(ss, axis=0)
    ct = jnp.transpose(ctab)[:, 0:NUM_FREQ]
    st = jnp.transpose(stab)[:, 0:NUM_FREQ]
    emb = a_ref[...] * ct + b_ref[...] * st
    out_ref[:, 0:EMBED_DIM] = (
        jnp.dot(emb, wt_ref[...], preferred_element_type=jnp.float32)
        + bias_ref[0:1, :]
    )
    out_ref[:, EMBED_DIM:TROW] = jnp.zeros(
        (ROW_BLOCK, TROW - EMBED_DIM), jnp.float32)


def _build_table(a_n, b_n, wt, bias_blk):
    grid = VOCAB // ROW_BLOCK
    return pl.pallas_call(
        _tc_table_body,
        grid=(grid,),
        in_specs=[
            pl.BlockSpec((ROW_BLOCK, NUM_FREQ), lambda i: (i, 0)),
            pl.BlockSpec((ROW_BLOCK, NUM_FREQ), lambda i: (i, 0)),
            pl.BlockSpec((NUM_FREQ, EMBED_DIM), lambda i: (0, 0)),
            pl.BlockSpec((8, EMBED_DIM), lambda i: (0, 0)),
        ],
        out_specs=pl.BlockSpec((ROW_BLOCK, TROW), lambda i: (i, 0)),
        out_shape=jax.ShapeDtypeStruct((VOCAB, TROW), jnp.float32),
    )(a_n, b_n, wt, bias_blk)


def _sc_gather(ids2d, table):
    n_rows = ids2d.shape[0]
    ntok = n_rows * CHUNK
    info = plsc.get_sparse_core_info()
    nc, ns = info.num_cores, info.num_subcores
    nw = nc * ns
    rows_per_w = n_rows // nw
    outer = rows_per_w // NBUF

    mesh = plsc.VectorSubcoreMesh(core_axis_name="c", subcore_axis_name="s")

    @functools.partial(
        pl.kernel,
        mesh=mesh,
        out_type=jax.ShapeDtypeStruct((ntok, TROW), jnp.float32),
        scratch_types=[
            pltpu.VMEM((rows_per_w, CHUNK), jnp.int32),
            pltpu.VMEM((NBUF, CHUNK, TROW), jnp.float32),
            [pltpu.SemaphoreType.DMA] * NBUF,
            [pltpu.SemaphoreType.DMA] * NBUF,
        ],
    )
    def k(ids_hbm, table_hbm, out_hbm, idx_v, rows_v, gsems, wsems):
        wid = lax.axis_index("s") * nc + lax.axis_index("c")
        row0 = wid * rows_per_w
        tok0 = row0 * CHUNK
        pltpu.sync_copy(ids_hbm.at[pl.ds(row0, rows_per_w)], idx_v)

        def fire(s, buf):
            pltpu.async_copy(table_hbm.at[idx_v.at[s]],
                             rows_v.at[buf], gsems[buf])

        def drain_g(buf):
            pltpu.make_async_copy(table_hbm.at[idx_v.at[0]],
                                  rows_v.at[buf], gsems[buf]).wait()

        def write(s, buf):
            pltpu.async_copy(rows_v.at[buf],
                             out_hbm.at[pl.ds(tok0 + s * CHUNK, CHUNK)],
                             wsems[buf])

        def drain_w(buf):
            pltpu.make_async_copy(rows_v.at[buf],
                                  out_hbm.at[pl.ds(tok0, CHUNK)],
                                  wsems[buf]).wait()

        for b in range(NBUF - 1):
            fire(b, b)

        def body(h, carry):
            for b in range(NBUF):
                s = NBUF * h + b
                nxt = s + NBUF - 1
                nbuf = (b + NBUF - 1) % NBUF

                @pl.when(jnp.logical_and(nxt >= NBUF, nxt < rows_per_w))
                def _():
                    drain_w(nbuf)

                @pl.when(nxt < rows_per_w)
                def _():
                    fire(nxt, nbuf)
                drain_g(b)
                write(s, b)
            return carry

        lax.fori_loop(0, outer, body, 0)
        for b in range(NBUF):
            drain_w(b)

    return k(ids2d, table)


def kernel(token_ids, a_n, b_n, W, b):
    B, S = token_ids.shape
    wt = W.T
    bias_blk = jnp.broadcast_to(b.reshape(1, EMBED_DIM), (8, EMBED_DIM))
    table = _build_table(a_n, b_n, wt, bias_blk)
    ids2d = token_ids.reshape(-1, CHUNK).astype(jnp.int32)
    out128 = _sc_gather(ids2d, table)
    return out128[:, 0:EMBED_DIM].reshape(B, S, EMBED_DIM)


# speedup vs baseline: 1.0428x; 1.0089x over previous
import functools
import math

import jax
import jax.numpy as jnp
from jax import lax
from jax.experimental import pallas as pl
from jax.experimental.pallas import tpu as pltpu
from jax.experimental.pallas import tpu_sc as plsc

VOCAB = 100000
NUM_FREQ = 50
EMBED_DIM = 64
TROW = 128
ROW_BLOCK = 10000

CHUNK = 128
NBUF = 4


FGROUPS = 7


def _tc_table_body(a_ref, b_ref, wt_ref, bias_ref, out_ref):
    i = pl.program_id(0)
    vf = (lax.broadcasted_iota(jnp.int32, (8, ROW_BLOCK), 1)
          + i * ROW_BLOCK).astype(jnp.float32)
    phi = (2.0 * math.pi) * (vf / float(VOCAB))
    f0 = (lax.broadcasted_iota(jnp.int32, (8, ROW_BLOCK), 0) + 1
          ).astype(jnp.float32)
    arg = f0 * phi
    c = jnp.cos(arg)
    s = jnp.sin(arg)
    cr = jnp.cos(8.0 * phi)
    sr = jnp.sin(8.0 * phi)
    cs, ss = [c], [s]
    for _ in range(FGROUPS - 1):
        c, s = c * cr - s * sr, c * sr + s * cr
        cs.append(c)
        ss.append(s)
    ctab = jnp.concatenate(cs, axis=0)
    stab = jnp.concatenate(ss, axis=0)
    ct = jnp.transpose(ctab)[:, 0:NUM_FREQ]
    st = jnp.transpose(stab)[:, 0:NUM_FREQ]
    emb = a_ref[...] * ct + b_ref[...] * st
    out_ref[:, 0:EMBED_DIM] = (
        jnp.dot(emb, wt_ref[...], preferred_element_type=jnp.float32)
        + bias_ref[0:1, :]
    )
    out_ref[:, EMBED_DIM:TROW] = jnp.zeros(
        (ROW_BLOCK, TROW - EMBED_DIM), jnp.float32)


def _build_table(a_n, b_n, wt, bias_blk):
    grid = VOCAB // ROW_BLOCK
    return pl.pallas_call(
        _tc_table_body,
        grid=(grid,),
        in_specs=[
            pl.BlockSpec((ROW_BLOCK, NUM_FREQ), lambda i: (i, 0)),
            pl.BlockSpec((ROW_BLOCK, NUM_FREQ), lambda i: (i, 0)),
            pl.BlockSpec((NUM_FREQ, EMBED_DIM), lambda i: (0, 0)),
            pl.BlockSpec((8, EMBED_DIM), lambda i: (0, 0)),
        ],
        out_specs=pl.BlockSpec((ROW_BLOCK, TROW), lambda i: (i, 0)),
        out_shape=jax.ShapeDtypeStruct((VOCAB, TROW), jnp.float32),
    )(a_n, b_n, wt, bias_blk)


def _sc_gather(ids2d, table):
    n_rows = ids2d.shape[0]
    ntok = n_rows * CHUNK
    info = plsc.get_sparse_core_info()
    nc, ns = info.num_cores, info.num_subcores
    nw = nc * ns
    rows_per_w = n_rows // nw
    outer = rows_per_w // NBUF

    mesh = plsc.VectorSubcoreMesh(core_axis_name="c", subcore_axis_name="s")

    @functools.partial(
        pl.kernel,
        mesh=mesh,
        out_type=jax.ShapeDtypeStruct((ntok, TROW), jnp.float32),
        scratch_types=[
            pltpu.VMEM((rows_per_w, CHUNK), jnp.int32),
            pltpu.VMEM((NBUF, CHUNK, TROW), jnp.float32),
            [pltpu.SemaphoreType.DMA] * NBUF,
            [pltpu.SemaphoreType.DMA] * NBUF,
        ],
    )
    def k(ids_hbm, table_hbm, out_hbm, idx_v, rows_v, gsems, wsems):
        wid = lax.axis_index("s") * nc + lax.axis_index("c")
        row0 = wid * rows_per_w
        tok0 = row0 * CHUNK
        pltpu.sync_copy(ids_hbm.at[pl.ds(row0, rows_per_w)], idx_v)

        def fire(s, buf):
            pltpu.async_copy(table_hbm.at[idx_v.at[s]],
                             rows_v.at[buf], gsems[buf])

        def drain_g(buf):
            pltpu.make_async_copy(table_hbm.at[idx_v.at[0]],
                                  rows_v.at[buf], gsems[buf]).wait()

        def write(s, buf):
            pltpu.async_copy(rows_v.at[buf],
                             out_hbm.at[pl.ds(tok0 + s * CHUNK, CHUNK)],
                             wsems[buf])

        def drain_w(buf):
            pltpu.make_async_copy(rows_v.at[buf],
                                  out_hbm.at[pl.ds(tok0, CHUNK)],
                                  wsems[buf]).wait()

        for b in range(NBUF - 1):
            fire(b, b)

        def body(h, carry):
            for b in range(NBUF):
                s = NBUF * h + b
                nxt = s + NBUF - 1
                nbuf = (b + NBUF - 1) % NBUF

                @pl.when(jnp.logical_and(nxt >= NBUF, nxt < rows_per_w))
                def _():
                    drain_w(nbuf)

                @pl.when(nxt < rows_per_w)
                def _():
                    fire(nxt, nbuf)
                drain_g(b)
                write(s, b)
            return carry

        lax.fori_loop(0, outer, body, 0)
        for b in range(NBUF):
            drain_w(b)

    return k(ids2d, table)


def kernel(token_ids, a_n, b_n, W, b):
    B, S = token_ids.shape
    wt = W.T
    bias_blk = jnp.broadcast_to(b.reshape(1, EMBED_DIM), (8, EMBED_DIM))
    table = _build_table(a_n, b_n, wt, bias_blk)
    ids2d = token_ids.reshape(-1, CHUNK).astype(jnp.int32)
    out128 = _sc_gather(ids2d, table)
    return out128[:, 0:EMBED_DIM].reshape(B, S, EMBED_DIM)
